# baseline (speedup 1.0000x reference)
import math

import numpy as np
import jax
import jax.numpy as jnp
from jax.experimental import pallas as pl
from jax.experimental.pallas import tpu as pltpu

B, S, H, R, K, A = 4, 2048, 2048, 64, 8, 32
HD = 64
SBLK = 512
NSTEPS = S // SBLK


def _hadamard_np(n):
    if n == 1:
        return np.array([[1.0]], dtype=np.float64)
    h = _hadamard_np(n // 2)
    top = np.concatenate([h, h], axis=1)
    bot = np.concatenate([h, -h], axis=1)
    return np.concatenate([top, bot], axis=0) / math.sqrt(n)


_HMAT_T = np.ascontiguousarray(_hadamard_np(HD).T.astype(np.float32))


def _fused_kernel(hid_ref, act_ref, hmt_ref, w1_ref, b1_ref, lng_ref, lnb_ref,
                  w2_ref, b2_ref, ml_ref, rs_ref, out_ref, acc_ref):
    i = pl.program_id(0)

    part = jnp.sum(hid_ref[...], axis=1)

    @pl.when(i == 0)
    def _init():
        acc_ref[...] = part

    @pl.when(i > 0)
    def _accum():
        acc_ref[...] += part

    @pl.when(i == NSTEPS - 1)
    def _finish():
        pooled = acc_ref[...] * (1.0 / S)
        h = jax.lax.dot_general(
            pooled, w1_ref[...], (((1,), (1,)), ((), ())),
            precision=jax.lax.Precision.HIGHEST,
            preferred_element_type=jnp.float32) + b1_ref[...]
        mu = jnp.mean(h, axis=-1, keepdims=True)
        var = jnp.mean((h - mu) ** 2, axis=-1, keepdims=True)
        h = (h - mu) * jax.lax.rsqrt(var + 1e-5) * lng_ref[...] + lnb_ref[...]
        h = h * 0.5 * (1.0 + jax.lax.erf(h * (1.0 / math.sqrt(2.0))))
        logits = jax.lax.dot_general(
            h, w2_ref[...], (((1,), (1,)), ((), ())),
            precision=jax.lax.Precision.HIGHEST,
            preferred_element_type=jnp.float32) + b2_ref[...]
        combined = logits + ml_ref[...]

        iota = jax.lax.broadcasted_iota(jnp.int32, (B, R), 1)
        avail = combined
        mask = jnp.zeros((B, R), jnp.float32)
        for _ in range(K):
            m = jnp.max(avail, axis=1, keepdims=True)
            is_max = avail == m
            idx = jnp.min(jnp.where(is_max, iota, R), axis=1, keepdims=True)
            sel = iota == idx
            mask = jnp.where(sel, 1.0, mask)
            avail = jnp.where(sel, -jnp.inf, avail)

        w = mask * rs_ref[...]

        hmt = hmt_ref[...]
        for b in range(B):
            mb = hmt * w[b].reshape(HD, 1)
            out_ref[b] = jax.lax.dot_general(
                act_ref[b], mb, (((1,), (0,)), ((), ())),
                precision=jax.lax.Precision.HIGHEST,
                preferred_element_type=jnp.float32)


def kernel(rank_activations, hidden_states, W1, b1, ln_g, ln_b, W2, b2, mask_logits, rank_scales):
    hmt = jnp.asarray(_HMAT_T)
    out = pl.pallas_call(
        _fused_kernel,
        grid=(NSTEPS,),
        in_specs=[
            pl.BlockSpec((B, SBLK, H), lambda i: (0, i, 0)),
            pl.BlockSpec((B, S, R), lambda i: (0, 0, 0)),
            pl.BlockSpec((HD, HD), lambda i: (0, 0)),
            pl.BlockSpec((A, H), lambda i: (0, 0)),
            pl.BlockSpec((1, A), lambda i: (0, 0)),
            pl.BlockSpec((1, A), lambda i: (0, 0)),
            pl.BlockSpec((1, A), lambda i: (0, 0)),
            pl.BlockSpec((R, A), lambda i: (0, 0)),
            pl.BlockSpec((1, R), lambda i: (0, 0)),
            pl.BlockSpec((1, R), lambda i: (0, 0)),
            pl.BlockSpec((1, R), lambda i: (0, 0)),
        ],
        out_specs=pl.BlockSpec((B, S, R), lambda i: (0, 0, 0)),
        out_shape=jax.ShapeDtypeStruct((B, S, R), jnp.float32),
        scratch_shapes=[pltpu.VMEM((B, H), jnp.float32)],
    )(
        hidden_states, rank_activations, hmt, W1,
        b1.reshape(1, A), ln_g.reshape(1, A), ln_b.reshape(1, A),
        W2, b2.reshape(1, R), mask_logits.reshape(1, R), rank_scales.reshape(1, R),
    )
    return out


# speedup vs baseline: 1.0861x; 1.0861x over previous
import math

import numpy as np
import jax
import jax.numpy as jnp
from jax.experimental import pallas as pl
from jax.experimental.pallas import tpu as pltpu

B, S, H, R, K, A = 4, 2048, 2048, 64, 8, 32
HD = 64
CH = 256
NCHUNK = (B * S) // CH
CPB = S // CH
NBUF = 8


def _hadamard_np(n):
    if n == 1:
        return np.array([[1.0]], dtype=np.float64)
    h = _hadamard_np(n // 2)
    top = np.concatenate([h, h], axis=1)
    bot = np.concatenate([h, -h], axis=1)
    return np.concatenate([top, bot], axis=0) / math.sqrt(n)


_HMAT_T = np.ascontiguousarray(_hadamard_np(HD).T.astype(np.float32))


def _fused_kernel(hid_ref, act_ref, hmt_ref, w1_ref, b1_ref, lng_ref, lnb_ref,
                  w2_ref, b2_ref, ml_ref, rs_ref, out_ref,
                  buf_ref, outv_ref, insem, outsem):
    def start_in(c):
        pltpu.make_async_copy(
            hid_ref.at[pl.ds(c * CH, CH), :], buf_ref.at[c % NBUF],
            insem.at[c % NBUF]).start()

    for c in range(NBUF):
        start_in(c)

    iota = jax.lax.broadcasted_iota(jnp.int32, (1, R), 1)

    for b in range(B):
        acc = None
        for j in range(CPB):
            c = b * CPB + j
            pltpu.make_async_copy(
                hid_ref.at[pl.ds(c * CH, CH), :], buf_ref.at[c % NBUF],
                insem.at[c % NBUF]).wait()
            part = jnp.sum(buf_ref[c % NBUF], axis=0, keepdims=True)
            acc = part if acc is None else acc + part
            if c + NBUF < NCHUNK:
                start_in(c + NBUF)

        pooled = acc * (1.0 / S)
        h = jax.lax.dot_general(
            pooled, w1_ref[...], (((1,), (1,)), ((), ())),
            precision=jax.lax.Precision.HIGHEST,
            preferred_element_type=jnp.float32) + b1_ref[...]
        mu = jnp.mean(h, axis=-1, keepdims=True)
        var = jnp.mean((h - mu) ** 2, axis=-1, keepdims=True)
        h = (h - mu) * jax.lax.rsqrt(var + 1e-5) * lng_ref[...] + lnb_ref[...]
        h = h * 0.5 * (1.0 + jax.lax.erf(h * (1.0 / math.sqrt(2.0))))
        logits = jax.lax.dot_general(
            h, w2_ref[...], (((1,), (1,)), ((), ())),
            precision=jax.lax.Precision.HIGHEST,
            preferred_element_type=jnp.float32) + b2_ref[...]
        combined = logits + ml_ref[...]

        avail = combined
        mask = jnp.zeros((1, R), jnp.float32)
        for _ in range(K):
            m = jnp.max(avail, axis=1, keepdims=True)
            is_max = avail == m
            idx = jnp.min(jnp.where(is_max, iota, R), axis=1, keepdims=True)
            sel = iota == idx
            mask = jnp.where(sel, 1.0, mask)
            avail = jnp.where(sel, -jnp.inf, avail)

        w = mask * rs_ref[...]

        scaled = act_ref[b] * w
        outv_ref[b] = jax.lax.dot_general(
            scaled, hmt_ref[...], (((1,), (0,)), ((), ())),
            precision=jax.lax.Precision.HIGHEST,
            preferred_element_type=jnp.float32)
        pltpu.make_async_copy(outv_ref.at[b], out_ref.at[b], outsem.at[b]).start()

    for b in range(B):
        pltpu.make_async_copy(outv_ref.at[b], out_ref.at[b], outsem.at[b]).wait()


def kernel(rank_activations, hidden_states, W1, b1, ln_g, ln_b, W2, b2, mask_logits, rank_scales):
    hmt = jnp.asarray(_HMAT_T)
    out = pl.pallas_call(
        _fused_kernel,
        in_specs=[
            pl.BlockSpec(memory_space=pl.ANY),
            pl.BlockSpec((B, S, R), lambda: (0, 0, 0)),
            pl.BlockSpec((HD, HD), lambda: (0, 0)),
            pl.BlockSpec((A, H), lambda: (0, 0)),
            pl.BlockSpec((1, A), lambda: (0, 0)),
            pl.BlockSpec((1, A), lambda: (0, 0)),
            pl.BlockSpec((1, A), lambda: (0, 0)),
            pl.BlockSpec((R, A), lambda: (0, 0)),
            pl.BlockSpec((1, R), lambda: (0, 0)),
            pl.BlockSpec((1, R), lambda: (0, 0)),
            pl.BlockSpec((1, R), lambda: (0, 0)),
        ],
        out_specs=pl.BlockSpec(memory_space=pl.ANY),
        out_shape=jax.ShapeDtypeStruct((B, S, R), jnp.float32),
        scratch_shapes=[
            pltpu.VMEM((NBUF, CH, H), jnp.float32),
            pltpu.VMEM((B, S, R), jnp.float32),
            pltpu.SemaphoreType.DMA((NBUF,)),
            pltpu.SemaphoreType.DMA((B,)),
        ],
    )(
        hidden_states.reshape(B * S, H), rank_activations, hmt, W1,
        b1.reshape(1, A), ln_g.reshape(1, A), ln_b.reshape(1, A),
        W2, b2.reshape(1, R), mask_logits.reshape(1, R), rank_scales.reshape(1, R),
    )
    return out
